# baseline (speedup 1.0000x reference)
import functools

import jax
import jax.numpy as jnp
from jax import lax
from jax.experimental import pallas as pl
from jax.experimental.pallas import tpu as pltpu
from jax.experimental.pallas import tpu_sc as plsc

NUM_CORES = 2
NUM_SUBCORES = 16
IDX_CHUNK = 128
LANE_TILES = 7813
FULL_TILES = 7812
TAIL = 64


def _retile_sc(table3):
    _, DPC, V = table3.shape
    n_rows = NUM_CORES * LANE_TILES * DPC
    per_tec = FULL_TILES // NUM_SUBCORES
    burst = 8

    mesh = plsc.VectorSubcoreMesh(core_axis_name="c", subcore_axis_name="s")

    @functools.partial(
        pl.kernel,
        mesh=mesh,
        out_type=jax.ShapeDtypeStruct((n_rows, 128), jnp.float32),
        scratch_types=[pltpu.SemaphoreType.DMA],
    )
    def retile_kernel(t3_hbm, lin_hbm, sem):
        core = lax.axis_index("c")
        tid = lax.axis_index("s")
        row0 = core * LANE_TILES * DPC

        def copy_tile(j):
            return pltpu.make_async_copy(
                t3_hbm.at[core, :, pl.ds(j * 128, 128)],
                lin_hbm.at[pl.ds(row0 + j * DPC, DPC), :],
                sem,
            )

        def body(k, carry):
            cps = [copy_tile(tid + (k * burst + u) * NUM_SUBCORES)
                   for u in range(burst)]
            for cp in cps:
                cp.start()
            for cp in cps:
                cp.wait()
            return carry

        lax.fori_loop(0, per_tec // burst, body, 0)

        @pl.when(tid < FULL_TILES - per_tec * NUM_SUBCORES)
        def _leftover():
            cp = copy_tile(per_tec * NUM_SUBCORES + tid)
            cp.start()
            cp.wait()

        @pl.when(tid < DPC)
        def _tail():
            cp = pltpu.make_async_copy(
                t3_hbm.at[core, tid, pl.ds(FULL_TILES * 128, TAIL)],
                lin_hbm.at[row0 + FULL_TILES * DPC + tid, pl.ds(0, TAIL)],
                sem,
            )
            cp.start()
            cp.wait()

    return retile_kernel(table3)


def _gather_sc(mols, lin_flat, D, V):
    B = mols.shape[0]
    DPC = D // NUM_CORES
    half = B // 2
    n_chunks = half // IDX_CHUNK

    mesh = plsc.VectorSubcoreMesh(core_axis_name="c", subcore_axis_name="s")

    @functools.partial(
        pl.kernel,
        mesh=mesh,
        out_type=jax.ShapeDtypeStruct((D * B,), jnp.float32),
        scratch_types=[
            pltpu.VMEM((half,), jnp.int32),
            pltpu.VMEM((half,), jnp.float32),
            pltpu.SemaphoreType.DMA,
        ],
    )
    def gather_kernel(idx_hbm, lin_hbm, mv_hbm, idx_v, val_v, sem):
        core = lax.axis_index("c")
        tid = lax.axis_index("s")
        r = lax.rem(tid, DPC)
        h = tid // DPC
        d = core * DPC + r
        pltpu.sync_copy(idx_hbm.at[pl.ds(h * half, half)], idx_v)
        base = core * LANE_TILES * DPC * 128 + r * 128
        for k in range(half // 16):
            sl = pl.ds(k * 16, 16)
            iv = idx_v[sl]
            idx_v[sl] = base + ((iv >> 7) << 10) + (iv & 127)
        copies = [
            pltpu.make_async_copy(
                lin_hbm.at[idx_v.at[pl.ds(k * IDX_CHUNK, IDX_CHUNK)]],
                val_v.at[pl.ds(k * IDX_CHUNK, IDX_CHUNK)],
                sem,
            )
            for k in range(n_chunks)
        ]
        for cp in copies:
            cp.start()
        for cp in copies:
            cp.wait()
        pltpu.sync_copy(val_v, mv_hbm.at[pl.ds(d * B + h * half, half)])

    return gather_kernel(mols, lin_flat)


def _scores_tc(mv_t, task_table):
    D, B = mv_t.shape
    T = task_table.shape[0]
    BB = 2048

    def mm_kernel(tt_ref, mv_ref, out_ref):
        out_ref[...] = jnp.dot(
            tt_ref[...], mv_ref[...], preferred_element_type=jnp.float32
        )

    return pl.pallas_call(
        mm_kernel,
        grid=(B // BB,),
        in_specs=[
            pl.BlockSpec((T, D), lambda i: (0, 0)),
            pl.BlockSpec((D, BB), lambda i: (0, i)),
        ],
        out_specs=pl.BlockSpec((T, BB), lambda i: (0, i)),
        out_shape=jax.ShapeDtypeStruct((T, B), jnp.float32),
    )(task_table, mv_t)


def kernel(mols, mol_table, task_table):
    V, D = mol_table.shape
    B = mols.shape[0]
    mols = mols.astype(jnp.int32)
    table3 = mol_table.T.reshape(NUM_CORES, D // NUM_CORES, V)
    lin2 = _retile_sc(table3)
    lin_flat = lin2.reshape(lin2.shape[0] * 128)
    mv_flat = _gather_sc(mols, lin_flat, D, V)
    mv_t = mv_flat.reshape(D, B)
    return _scores_tc(mv_t, task_table).T


# speedup vs baseline: 13.6642x; 13.6642x over previous
import functools

import jax
import jax.numpy as jnp
from jax import lax
from jax.experimental import pallas as pl
from jax.experimental.pallas import tpu as pltpu
from jax.experimental.pallas import tpu_sc as plsc

NUM_CORES = 2
NUM_SUBCORES = 16
IDX_CHUNK = 128
LANE_TILES = 7813
FULL_TILES = 7812
TAIL = 64


K_TILES = 128
BLOCKS_PER_CORE = (LANE_TILES + K_TILES - 1) // K_TILES
CORE_ROWS = BLOCKS_PER_CORE * K_TILES * 8


def _retile_tc(table3):
    C, DPC, V = table3.shape
    RB = K_TILES * DPC

    def rk(x_ref, o_ref):
        for j in range(K_TILES):
            o_ref[j * DPC:(j + 1) * DPC, :] = x_ref[0, :, j * 128:(j + 1) * 128]

    return pl.pallas_call(
        rk,
        grid=(C, BLOCKS_PER_CORE),
        in_specs=[pl.BlockSpec((1, DPC, K_TILES * 128), lambda c, jb: (c, 0, jb))],
        out_specs=pl.BlockSpec(
            (RB, 128), lambda c, jb: (c * BLOCKS_PER_CORE + jb, 0)
        ),
        out_shape=jax.ShapeDtypeStruct((C * CORE_ROWS, 128), jnp.float32),
    )(table3)


def _gather_sc(mols, lin_flat, D, V):
    B = mols.shape[0]
    DPC = D // NUM_CORES
    half = B // 2
    n_chunks = half // IDX_CHUNK

    mesh = plsc.VectorSubcoreMesh(core_axis_name="c", subcore_axis_name="s")

    @functools.partial(
        pl.kernel,
        mesh=mesh,
        out_type=jax.ShapeDtypeStruct((D * B,), jnp.float32),
        scratch_types=[
            pltpu.VMEM((half,), jnp.int32),
            pltpu.VMEM((half,), jnp.float32),
            pltpu.SemaphoreType.DMA,
        ],
    )
    def gather_kernel(idx_hbm, lin_hbm, mv_hbm, idx_v, val_v, sem):
        core = lax.axis_index("c")
        tid = lax.axis_index("s")
        r = lax.rem(tid, DPC)
        h = tid // DPC
        d = core * DPC + r
        pltpu.sync_copy(idx_hbm.at[pl.ds(h * half, half)], idx_v)
        base = core * CORE_ROWS * 128 + r * 128
        for k in range(half // 16):
            sl = pl.ds(k * 16, 16)
            iv = idx_v[sl]
            idx_v[sl] = base + ((iv >> 7) << 10) + (iv & 127)
        copies = [
            pltpu.make_async_copy(
                lin_hbm.at[idx_v.at[pl.ds(k * IDX_CHUNK, IDX_CHUNK)]],
                val_v.at[pl.ds(k * IDX_CHUNK, IDX_CHUNK)],
                sem,
            )
            for k in range(n_chunks)
        ]
        for cp in copies:
            cp.start()
        for cp in copies:
            cp.wait()
        pltpu.sync_copy(val_v, mv_hbm.at[pl.ds(d * B + h * half, half)])

    return gather_kernel(mols, lin_flat)


def _scores_tc(mv_t, task_table):
    D, B = mv_t.shape
    T = task_table.shape[0]
    BB = 2048

    def mm_kernel(tt_ref, mv_ref, out_ref):
        out_ref[...] = jnp.dot(
            tt_ref[...], mv_ref[...], preferred_element_type=jnp.float32
        )

    return pl.pallas_call(
        mm_kernel,
        grid=(B // BB,),
        in_specs=[
            pl.BlockSpec((T, D), lambda i: (0, 0)),
            pl.BlockSpec((D, BB), lambda i: (0, i)),
        ],
        out_specs=pl.BlockSpec((T, BB), lambda i: (0, i)),
        out_shape=jax.ShapeDtypeStruct((T, B), jnp.float32),
    )(task_table, mv_t)


def kernel(mols, mol_table, task_table):
    V, D = mol_table.shape
    B = mols.shape[0]
    mols = mols.astype(jnp.int32)
    table3 = mol_table.T.reshape(NUM_CORES, D // NUM_CORES, V)
    lin2 = _retile_tc(table3)
    lin_flat = lin2.reshape(lin2.shape[0] * 128)
    mv_flat = _gather_sc(mols, lin_flat, D, V)
    mv_t = mv_flat.reshape(D, B)
    return _scores_tc(mv_t, task_table).T


# speedup vs baseline: 16.7351x; 1.2247x over previous
import functools

import jax
import jax.numpy as jnp
from jax import lax
from jax.experimental import pallas as pl
from jax.experimental.pallas import tpu as pltpu
from jax.experimental.pallas import tpu_sc as plsc

NUM_CORES = 2
NUM_SUBCORES = 16
IDX_CHUNK = 128
LANE_TILES = 7813
FULL_TILES = 7812
TAIL = 64


K_TILES = 256
BLOCKS_PER_CORE = (LANE_TILES + K_TILES - 1) // K_TILES
CORE_ROWS = BLOCKS_PER_CORE * K_TILES * 8


def _retile_tc(table3):
    C, DPC, V = table3.shape
    RB = K_TILES * DPC

    def rk(x_ref, o_ref):
        for j in range(K_TILES):
            o_ref[j * DPC:(j + 1) * DPC, :] = x_ref[0, :, j * 128:(j + 1) * 128]

    return pl.pallas_call(
        rk,
        grid=(C, BLOCKS_PER_CORE),
        in_specs=[pl.BlockSpec((1, DPC, K_TILES * 128), lambda c, jb: (c, 0, jb))],
        out_specs=pl.BlockSpec(
            (RB, 128), lambda c, jb: (c * BLOCKS_PER_CORE + jb, 0)
        ),
        out_shape=jax.ShapeDtypeStruct((C * CORE_ROWS, 128), jnp.float32),
    )(table3)


def _gather_sc(mols, lin_flat, D, V):
    B = mols.shape[0]
    DPC = D // NUM_CORES
    half = B // 2
    n_chunks = half // IDX_CHUNK

    mesh = plsc.VectorSubcoreMesh(core_axis_name="c", subcore_axis_name="s")

    @functools.partial(
        pl.kernel,
        mesh=mesh,
        out_type=jax.ShapeDtypeStruct((D * B,), jnp.float32),
        scratch_types=[
            pltpu.VMEM((half,), jnp.int32),
            pltpu.VMEM((half,), jnp.float32),
            pltpu.SemaphoreType.DMA,
        ],
    )
    def gather_kernel(idx_hbm, lin_hbm, mv_hbm, idx_v, val_v, sem):
        core = lax.axis_index("c")
        tid = lax.axis_index("s")
        r = lax.rem(tid, DPC)
        h = tid // DPC
        d = core * DPC + r
        pltpu.sync_copy(idx_hbm.at[pl.ds(h * half, half)], idx_v)
        base = core * CORE_ROWS * 128 + r * 128
        for k in range(half // 16):
            sl = pl.ds(k * 16, 16)
            iv = idx_v[sl]
            idx_v[sl] = base + ((iv >> 7) << 10) + (iv & 127)
        copies = [
            pltpu.make_async_copy(
                lin_hbm.at[idx_v.at[pl.ds(k * IDX_CHUNK, IDX_CHUNK)]],
                val_v.at[pl.ds(k * IDX_CHUNK, IDX_CHUNK)],
                sem,
            )
            for k in range(n_chunks)
        ]
        for cp in copies:
            cp.start()
        for cp in copies:
            cp.wait()
        pltpu.sync_copy(val_v, mv_hbm.at[pl.ds(d * B + h * half, half)])

    return gather_kernel(mols, lin_flat)


def _scores_tc(mv_t, task_table):
    D, B = mv_t.shape
    T = task_table.shape[0]
    BB = 2048

    def mm_kernel(tt_ref, mv_ref, out_ref):
        out_ref[...] = jnp.dot(
            tt_ref[...], mv_ref[...], preferred_element_type=jnp.float32
        )

    return pl.pallas_call(
        mm_kernel,
        grid=(B // BB,),
        in_specs=[
            pl.BlockSpec((T, D), lambda i: (0, 0)),
            pl.BlockSpec((D, BB), lambda i: (0, i)),
        ],
        out_specs=pl.BlockSpec((T, BB), lambda i: (0, i)),
        out_shape=jax.ShapeDtypeStruct((T, B), jnp.float32),
    )(task_table, mv_t)


def kernel(mols, mol_table, task_table):
    V, D = mol_table.shape
    B = mols.shape[0]
    mols = mols.astype(jnp.int32)
    table3 = mol_table.T.reshape(NUM_CORES, D // NUM_CORES, V)
    lin2 = _retile_tc(table3)
    lin_flat = lin2.reshape(lin2.shape[0] * 128)
    mv_flat = _gather_sc(mols, lin_flat, D, V)
    mv_t = mv_flat.reshape(D, B)
    return _scores_tc(mv_t, task_table).T
